# baseline (speedup 1.0000x reference)
import functools

import jax
import jax.numpy as jnp
from jax import lax
from jax.experimental import pallas as pl
from jax.experimental.pallas import tpu as pltpu
from jax.experimental.pallas import tpu_sc as plsc


def _ids_body(zt_ref, cbt_ref, ids3_ref):
    cbt = cbt_ref[...]
    bias = 0.5 * jnp.sum(cbt * cbt, axis=0)[:, None]
    n_b = zt_ref.shape[0]
    n_ch, CH = ids3_ref.shape[2], ids3_ref.shape[3]
    for b in range(n_b):
        zbt = zt_ref[b]
        score = lax.dot_general(
            cbt, zbt, (((0,), (0,)), ((), ())),
            preferred_element_type=jnp.float32)
        score = score - bias
        ids = jnp.argmax(score, axis=0).astype(jnp.int32)
        ids3_ref[b, 0] = ids.reshape(n_ch, CH)


def _tc_ids(zt, cbt):
    B, D, T = zt.shape
    K = cbt.shape[1]
    n_ch, CH = T // 128, 128
    NB = 8
    ids3 = pl.pallas_call(
        _ids_body,
        grid=(B // NB,),
        in_specs=[
            pl.BlockSpec((NB, D, T), lambda i: (i, 0, 0)),
            pl.BlockSpec((D, K), lambda i: (0, 0)),
        ],
        out_specs=pl.BlockSpec((NB, 1, n_ch, CH), lambda i: (i, 0, 0, 0)),
        out_shape=jax.ShapeDtypeStruct((B, 1, n_ch, CH), jnp.int32),
    )(zt, cbt)
    ids3 = ids3.reshape(B, n_ch, CH)
    return ids3, ids3.reshape(B, T)


def _sc_gather_t(cbt, ids, B, D, T):
    K = cbt.shape[1]
    mesh = plsc.VectorSubcoreMesh(core_axis_name="c", subcore_axis_name="s")

    @functools.partial(
        pl.kernel,
        mesh=mesh,
        compiler_params=pltpu.CompilerParams(
            use_tc_tiling_on_sc=False, needs_layout_passes=False),
        out_type=jax.ShapeDtypeStruct((B, D // 8, T // 128, 8, 128),
                                      jnp.float32),
        scratch_types=[
            pltpu.VMEM((D, K), jnp.float32),
            pltpu.VMEM((T,), jnp.int32),
            pltpu.VMEM((D // 8, T // 128, 8, 128), jnp.float32),
            pltpu.SemaphoreType.DMA,
            pltpu.SemaphoreType.DMA,
            pltpu.SemaphoreType.DMA,
        ],
    )
    def k(cb_hbm, idx_hbm, out_hbm, cb_v, idx_v, zq_v, sem_cb, sem_ix,
          sem_out):
        nc = lax.axis_size("c")
        wid = lax.axis_index("s") * nc + lax.axis_index("c")
        cb_cp = pltpu.async_copy(cb_hbm, cb_v, sem_cb)
        ix_cp = pltpu.async_copy(idx_hbm.at[wid], idx_v, sem_ix)
        cb_cp.wait()
        ix_cp.wait()

        out_cps = []
        for p in range(D // 8):

            @plsc.parallel_loop(0, T // 16, step=1, unroll=8)
            def band(i, p=p):
                ids16 = idx_v[pl.ds(i * 16, 16)]
                tb = i // 8
                lo = (i % 8) * 16
                for dd in range(8):
                    dvec = jnp.full((16,), p * 8 + dd, jnp.int32)
                    vals = plsc.load_gather(cb_v, [dvec, ids16])
                    zq_v[p, tb, dd, pl.ds(lo, 16)] = vals

            out_cps.append(
                pltpu.async_copy(zq_v.at[p], out_hbm.at[wid, p], sem_out))
        for cp in out_cps:
            cp.wait()

    return k(cbt, ids)


def kernel(z, codebook):
    B, T, D = z.shape
    K = codebook.shape[0]
    zt = jnp.swapaxes(z, 1, 2)
    cbt = codebook.T
    ids3, ids = _tc_ids(zt, cbt)
    zq5 = _sc_gather_t(cbt, ids3.reshape(B, T), B, D, T)
    zq_t = zq5.transpose(0, 1, 3, 2, 4).reshape(B, D, T)
    return jnp.swapaxes(zq_t, 1, 2), ids
